# baseline (speedup 1.0000x reference)
import jax
import jax.numpy as jnp
import numpy as np
from jax.experimental import pallas as pl

_TAU = 0.9
_N_ITERS = 28


def _topk_mask_kernel(z_ref, out_ref):
    z = z_ref[:]
    m = jnp.max(z, axis=1, keepdims=True)
    e = jnp.exp(z - m)
    s = jnp.sum(e, axis=1, keepdims=True)
    target = _TAU * s

    lo = jax.lax.bitcast_convert_type(
        s * np.float32((1.0 - _TAU) / 65536.0), jnp.int32
    )
    hi = jnp.zeros_like(lo) + np.int32(0x3F800001)

    def body(_, carry):
        lo, hi = carry
        mid = (lo & hi) + ((lo ^ hi) >> 1)
        mid_f = jax.lax.bitcast_convert_type(mid, jnp.float32)
        g = jnp.sum(jnp.where(e >= mid_f, e, 0.0), axis=1, keepdims=True)
        pred = g >= target
        return jnp.where(pred, mid, lo), jnp.where(pred, hi, mid)

    lo, hi = jax.lax.fori_loop(0, _N_ITERS, body, (lo, hi))

    lo_f = jax.lax.bitcast_convert_type(lo, jnp.float32)
    theta = jnp.max(jnp.where(e <= lo_f, e, 0.0), axis=1, keepdims=True)
    out_ref[:] = jnp.where(e >= theta, jnp.maximum(z, 0.0), 0.0)


@jax.jit
def kernel(z):
    rows, n = z.shape
    block_rows = 64
    grid = (rows // block_rows,)
    return pl.pallas_call(
        _topk_mask_kernel,
        grid=grid,
        in_specs=[pl.BlockSpec((block_rows, n), lambda i: (i, 0))],
        out_specs=pl.BlockSpec((block_rows, n), lambda i: (i, 0)),
        out_shape=jax.ShapeDtypeStruct((rows, n), jnp.float32),
    )(z)
